# baseline (speedup 1.0000x reference)
import functools

import jax
import jax.numpy as jnp
from jax import lax
from jax.experimental import pallas as pl
from jax.experimental.pallas import tpu as pltpu
from jax.experimental.pallas import tpu_sc as plsc

_B = 4096
_L = 200
_D = 64
_V = 1000000
_N = _B * _L
_NC = 2
_NS = 16
_NW = _NC * _NS
_LANES = 16

_VT = _V // 128
_ASLOT = 246

_CHB = 256
_UPW = _N // _NW // _CHB
_BT = _B // 128
_PW = _N // _NW


def _lane_sum(vec):
    s = vec[0]
    for i in range(1, _LANES):
        s = s + vec[i]
    return s


_mesh = plsc.VectorSubcoreMesh(core_axis_name="c", subcore_axis_name="s")


@functools.partial(
    pl.kernel,
    mesh=_mesh,
    out_type=jax.ShapeDtypeStruct((_V * 128,), jnp.float32),
    scratch_types=[
        pltpu.VMEM((2, 8, 8, 128), jnp.float32),
        pltpu.VMEM((128 * 128,), jnp.float32),
        pltpu.VMEM((128 * 128,), jnp.float32),
        pltpu.VMEM((64 * 128,), jnp.float32),
        pltpu.SemaphoreType.DMA,
        pltpu.SemaphoreType.DMA,
        pltpu.SemaphoreType.DMA,
        pltpu.SemaphoreType.DMA,
    ],
    compiler_params=pltpu.CompilerParams(
        use_tc_tiling_on_sc=True, needs_layout_passes=False
    ),
)
def _table_rowmajor(
    tt_hbm, tail_hbm, out_hbm, inb, outb0, outb1, tailv, is0, is1, os0, os1
):
    wid = lax.axis_index("s") * _NC + lax.axis_index("c")
    lane = lax.iota(jnp.int32, _LANES)
    lane128 = lane * 128

    def ua(slot):
        return jnp.minimum(slot * _NW + wid, _VT - 1)

    def fire_in(slot, buf):
        vt = ua(slot)
        cps = []
        for dt in range(8):
            cps.append(
                pltpu.make_async_copy(
                    tt_hbm.at[pl.ds(dt * 8, 8), pl.ds(vt * 128, 128)],
                    inb.at[buf, dt],
                    is0 if buf == 0 else is1,
                )
            )
        return cps

    def transpose_unit(buf, outb):
        def vr_body(vr, _):
            basev = lane128 + vr * (16 * 128)
            for d in range(_D):
                v = inb[buf, d // 8, d % 8, pl.ds(vr * 16, 16)]
                plsc.store_scatter(outb, [basev + d], v)
            return 0

        lax.fori_loop(0, 8, vr_body, 0)

    @pl.when(wid == 0)
    def _():
        pltpu.sync_copy(tail_hbm, tailv)
        pltpu.sync_copy(tailv, out_hbm.at[pl.ds(_VT * 16384, 64 * 128)])

    for cp in fire_in(0, 0):
        cp.start()
    for cp in fire_in(1, 1):
        cp.start()

    def pair(k, _):
        for buf, outb, osem in ((0, outb0, os0), (1, outb1, os1)):
            slot = 2 * k + buf
            vt = ua(slot)
            for cp in fire_in(slot, buf):
                cp.wait()
            transpose_unit(buf, outb)

            @pl.when(k >= 1)
            def _(outb=outb, osem=osem, slot=slot):
                pv = ua(slot - 2)
                pltpu.make_async_copy(
                    outb, out_hbm.at[pl.ds(pv * 16384, 16384)], osem
                ).wait()

            pltpu.async_copy(outb, out_hbm.at[pl.ds(vt * 16384, 16384)], osem)

            @pl.when(k < (_ASLOT // 2) - 1)
            def _(slot=slot, buf=buf):
                for cp in fire_in(slot + 2, buf):
                    cp.start()
        return 0

    lax.fori_loop(0, _ASLOT // 2, pair, 0)

    pltpu.make_async_copy(
        outb0, out_hbm.at[pl.ds(ua(_ASLOT - 2) * 16384, 16384)], os0
    ).wait()
    pltpu.make_async_copy(
        outb1, out_hbm.at[pl.ds(ua(_ASLOT - 1) * 16384, 16384)], os1
    ).wait()


def _zero_fixup(idx_v, rows, c_base, flat_base):
    lane = lax.iota(jnp.int32, _LANES)
    zeros = jnp.zeros((_LANES,), jnp.float32)

    def group(r, _):
        start = c_base + r * _LANES
        v = idx_v[pl.ds(start, _LANES)]
        pos = flat_base + r * _LANES + lane
        m = (v == 0) & (pos >= _B)
        hit = _lane_sum(jnp.where(m, jnp.int32(1), jnp.int32(0)))

        @pl.when(hit > 0)
        def _():
            for j in range(_LANES):
                vj = v[j]
                p = flat_base + r * _LANES + j

                @pl.when((vj == 0) & (p >= _B))
                def _(j=j):
                    row = r * _LANES + j
                    for k in range(_D // _LANES):
                        rows[row, pl.ds(k * _LANES, _LANES)] = zeros

        return 0

    lax.fori_loop(0, _CHB // _LANES, group, 0)


@functools.partial(
    pl.kernel,
    mesh=_mesh,
    out_type=jax.ShapeDtypeStruct((_L * 8 * _BT * 8 * 128,), jnp.float32),
    scratch_types=[
        pltpu.VMEM((_PW,), jnp.int32),
        pltpu.VMEM((_CHB, _D), jnp.float32),
        pltpu.VMEM((_CHB, _D), jnp.float32),
        pltpu.VMEM((_CHB * _D,), jnp.float32),
        pltpu.VMEM((_CHB * _D,), jnp.float32),
        pltpu.SemaphoreType.DMA,
        pltpu.SemaphoreType.DMA,
        pltpu.SemaphoreType.DMA,
        pltpu.SemaphoreType.DMA,
    ],
    compiler_params=pltpu.CompilerParams(
        use_tc_tiling_on_sc=False, needs_layout_passes=False
    ),
)
def _emb_gather(
    idx_hbm, table_hbm, out_hbm, idx_v, rows0, rows1, tr0, tr1, gs0, gs1, os0, os1
):
    wid = lax.axis_index("s") * _NC + lax.axis_index("c")
    base = wid * _PW
    lane = lax.iota(jnp.int32, _LANES)
    lane_hi = jnp.where(lane >= 8, jnp.int32(1), jnp.int32(0))
    lane_lo128 = lax.rem(lane, 8) * 128
    basek = [(2 * k + lane_hi) * 2048 + lane_lo128 for k in range(4)]

    pltpu.sync_copy(idx_hbm.at[pl.ds(base, _PW)], idx_v)

    def dbl(i, _):
        v = idx_v[pl.ds(i * _LANES, _LANES)]
        idx_v[pl.ds(i * _LANES, _LANES)] = v + v
        return 0

    lax.fori_loop(0, _PW // _LANES, dbl, 0)

    def gather_cp(i, rows, sem):
        return pltpu.make_async_copy(
            table_hbm.at[idx_v.at[pl.ds(i * _CHB, _CHB)]], rows, sem
        )

    def obase(i):
        u = wid * _UPW + i
        l = u // 16
        bc = lax.rem(u, 16)
        return l * (8 * _BT * 1024) + bc * 2048

    def out_cps(i, tr, osem):
        ob = obase(i)
        return [
            pltpu.make_async_copy(
                tr.at[pl.ds(g * 2048, 2048)],
                out_hbm.at[pl.ds(ob + g * (_BT * 1024), 2048)],
                osem,
            )
            for g in range(8)
        ]

    def transpose_unit(rows, tr):
        def rbody(r, _):
            roff = (r // 128) * 1024 + lax.rem(r, 128)
            for k in range(4):
                v = rows[r, pl.ds(k * 16, 16)]
                plsc.store_scatter(tr, [basek[k] + roff], v)
            return 0

        lax.fori_loop(0, _CHB, rbody, 0)

    gather_cp(0, rows0, gs0).start()
    gather_cp(1, rows1, gs1).start()

    def pair(k, _):
        for buf, rows, tr, gsem, osem in (
            (0, rows0, tr0, gs0, os0),
            (1, rows1, tr1, gs1, os1),
        ):
            i = 2 * k + buf
            gather_cp(i, rows, gsem).wait()

            c_base = i * _CHB

            def red(t, acc):
                v = idx_v[pl.ds(c_base + t * _LANES, _LANES)]
                return acc + jnp.where(v == 0, jnp.int32(1), jnp.int32(0))

            acc = lax.fori_loop(
                0, _CHB // _LANES, red, jnp.zeros((_LANES,), jnp.int32)
            )
            nzero = _lane_sum(acc)

            @pl.when(nzero > 0)
            def _(rows=rows, c_base=c_base, i=i):
                _zero_fixup(idx_v, rows, c_base, base + i * _CHB)

            @pl.when(k >= 1)
            def _(i=i, tr=tr, osem=osem):
                for cp in out_cps(i - 2, tr, osem):
                    cp.wait()

            transpose_unit(rows, tr)

            for cp in out_cps(i, tr, osem):
                cp.start()

            @pl.when(k < (_UPW // 2) - 1)
            def _(i=i, rows=rows, gsem=gsem):
                gather_cp(i + 2, rows, gsem).start()
        return 0

    lax.fori_loop(0, _UPW // 2, pair, 0)

    for cp in out_cps(_UPW - 2, tr0, os0):
        cp.wait()
    for cp in out_cps(_UPW - 1, tr1, os1):
        cp.wait()


def kernel(x, table):
    table_t = table.T
    tailp = jnp.pad(table[_VT * 128 :], ((0, 0), (0, 128 - _D))).reshape(-1)
    scratch = _table_rowmajor(table_t, tailp)
    table2 = scratch.reshape(2 * _V, _D)
    xt = x.T.reshape(_N).astype(jnp.int32)
    y = _emb_gather(xt, table2)
    y5 = y.reshape(_L, 8, _BT, 8, 128)
    return y5.transpose(2, 4, 0, 1, 3).reshape(_B, _L, _D)


# speedup vs baseline: 1.1442x; 1.1442x over previous
import functools

import jax
import jax.numpy as jnp
from jax import lax
from jax.experimental import pallas as pl
from jax.experimental.pallas import tpu as pltpu
from jax.experimental.pallas import tpu_sc as plsc

_B = 4096
_L = 200
_D = 64
_V = 1000000
_N = _B * _L
_NC = 2
_NS = 16
_NW = _NC * _NS
_LANES = 16

_VT = _V // 128
_ASLOT = 246

_CHB = 256
_UPW = _N // _NW // _CHB
_BT = _B // 128
_PW = _N // _NW


def _lane_sum(vec):
    s = vec[0]
    for i in range(1, _LANES):
        s = s + vec[i]
    return s


_mesh = plsc.VectorSubcoreMesh(core_axis_name="c", subcore_axis_name="s")


@functools.partial(
    pl.kernel,
    mesh=_mesh,
    out_type=jax.ShapeDtypeStruct((_V * 128,), jnp.float32),
    scratch_types=[
        pltpu.VMEM((2, _D, 129), jnp.float32),
        pltpu.VMEM((128 * 128,), jnp.float32),
        pltpu.VMEM((128 * 128,), jnp.float32),
        pltpu.VMEM((64 * 128,), jnp.float32),
        pltpu.SemaphoreType.DMA,
        pltpu.SemaphoreType.DMA,
        pltpu.SemaphoreType.DMA,
        pltpu.SemaphoreType.DMA,
    ],
    compiler_params=pltpu.CompilerParams(
        use_tc_tiling_on_sc=True, needs_layout_passes=False
    ),
)
def _table_rowmajor(
    tt_hbm, tail_hbm, out_hbm, inb, outb0, outb1, tailv, is0, is1, os0, os1
):
    wid = lax.axis_index("s") * _NC + lax.axis_index("c")
    lane = lax.iota(jnp.int32, _LANES)
    rowsk = [lane + k * 16 for k in range(4)]

    def ua(slot):
        return jnp.minimum(slot * _NW + wid, _VT - 1)

    def fire_in(slot, buf):
        vt = ua(slot)
        return [
            pltpu.make_async_copy(
                tt_hbm.at[:, pl.ds(vt * 128, 128)],
                inb.at[buf, :, pl.ds(0, 128)],
                is0 if buf == 0 else is1,
            )
        ]

    def transpose_unit(buf, outb):
        def v_body(v, _):
            colv = jnp.full((_LANES,), v, jnp.int32)
            for k in range(4):
                r = plsc.load_gather(inb.at[buf], [rowsk[k], colv])
                outb[pl.ds(v * 128 + k * 16, 16)] = r
            return 0

        lax.fori_loop(0, 128, v_body, 0)

    @pl.when(wid == 0)
    def _():
        pltpu.sync_copy(tail_hbm, tailv)
        pltpu.sync_copy(tailv, out_hbm.at[pl.ds(_VT * 16384, 64 * 128)])

    for cp in fire_in(0, 0):
        cp.start()
    for cp in fire_in(1, 1):
        cp.start()

    def pair(k, _):
        for buf, outb, osem in ((0, outb0, os0), (1, outb1, os1)):
            slot = 2 * k + buf
            vt = ua(slot)
            for cp in fire_in(slot, buf):
                cp.wait()
            transpose_unit(buf, outb)

            @pl.when(k >= 1)
            def _(outb=outb, osem=osem, slot=slot):
                pv = ua(slot - 2)
                pltpu.make_async_copy(
                    outb, out_hbm.at[pl.ds(pv * 16384, 16384)], osem
                ).wait()

            pltpu.async_copy(outb, out_hbm.at[pl.ds(vt * 16384, 16384)], osem)

            @pl.when(k < (_ASLOT // 2) - 1)
            def _(slot=slot, buf=buf):
                for cp in fire_in(slot + 2, buf):
                    cp.start()
        return 0

    lax.fori_loop(0, _ASLOT // 2, pair, 0)

    pltpu.make_async_copy(
        outb0, out_hbm.at[pl.ds(ua(_ASLOT - 2) * 16384, 16384)], os0
    ).wait()
    pltpu.make_async_copy(
        outb1, out_hbm.at[pl.ds(ua(_ASLOT - 1) * 16384, 16384)], os1
    ).wait()


def _zero_fixup(idx_v, rows, c_base, flat_base):
    lane = lax.iota(jnp.int32, _LANES)
    zeros = jnp.zeros((_LANES,), jnp.float32)

    def group(r, _):
        start = c_base + r * _LANES
        v = idx_v[pl.ds(start, _LANES)]
        pos = flat_base + r * _LANES + lane
        m = (v == 0) & (pos >= _B)
        hit = _lane_sum(jnp.where(m, jnp.int32(1), jnp.int32(0)))

        @pl.when(hit > 0)
        def _():
            for j in range(_LANES):
                vj = v[j]
                p = flat_base + r * _LANES + j

                @pl.when((vj == 0) & (p >= _B))
                def _(j=j):
                    row = r * _LANES + j
                    for k in range(_D // _LANES):
                        rows[row, pl.ds(k * _LANES, _LANES)] = zeros

        return 0

    lax.fori_loop(0, _CHB // _LANES, group, 0)


@functools.partial(
    pl.kernel,
    mesh=_mesh,
    out_type=jax.ShapeDtypeStruct((_L * 8 * _BT * 8, 128), jnp.float32),
    scratch_types=[
        pltpu.VMEM((_PW,), jnp.int32),
        pltpu.VMEM((_CHB, _D), jnp.float32),
        pltpu.VMEM((_CHB, _D), jnp.float32),
        pltpu.VMEM((_D, 257), jnp.float32),
        pltpu.VMEM((_D, 257), jnp.float32),
        pltpu.SemaphoreType.DMA,
        pltpu.SemaphoreType.DMA,
        pltpu.SemaphoreType.DMA,
        pltpu.SemaphoreType.DMA,
    ],
    compiler_params=pltpu.CompilerParams(
        use_tc_tiling_on_sc=False, needs_layout_passes=False
    ),
)
def _emb_gather(
    idx_hbm, table_hbm, out_hbm, idx_v, rows0, rows1, tr0, tr1, gs0, gs1, os0, os1
):
    wid = lax.axis_index("s") * _NC + lax.axis_index("c")
    base = wid * _PW
    lane = lax.iota(jnp.int32, _LANES)
    rowsk = [lane + k * 16 for k in range(4)]

    pltpu.sync_copy(idx_hbm.at[pl.ds(base, _PW)], idx_v)

    def dbl(i, _):
        v = idx_v[pl.ds(i * _LANES, _LANES)]
        idx_v[pl.ds(i * _LANES, _LANES)] = v + v
        return 0

    lax.fori_loop(0, _PW // _LANES, dbl, 0)

    def gather_cp(i, rows, sem):
        return pltpu.make_async_copy(
            table_hbm.at[idx_v.at[pl.ds(i * _CHB, _CHB)]], rows, sem
        )

    def out_cps(i, tr, osem):
        u = wid * _UPW + i
        l = u // 16
        bc = lax.rem(u, 16)
        cps = []
        for g in range(8):
            for bt in range(2):
                r0 = ((l * 8 + g) * _BT + bc * 2 + bt) * 8
                cps.append(
                    pltpu.make_async_copy(
                        tr.at[pl.ds(g * 8, 8), pl.ds(bt * 128, 128)],
                        out_hbm.at[pl.ds(r0, 8), :],
                        osem,
                    )
                )
        return cps

    def transpose_unit(rows, tr):
        def rbody(r, _):
            colv = jnp.full((_LANES,), r, jnp.int32)
            for k in range(4):
                v = rows[r, pl.ds(k * 16, 16)]
                plsc.store_scatter(tr, [rowsk[k], colv], v)
            return 0

        lax.fori_loop(0, _CHB, rbody, 0)

    gather_cp(0, rows0, gs0).start()
    gather_cp(1, rows1, gs1).start()

    def pair(k, _):
        for buf, rows, tr, gsem, osem in (
            (0, rows0, tr0, gs0, os0),
            (1, rows1, tr1, gs1, os1),
        ):
            i = 2 * k + buf
            gather_cp(i, rows, gsem).wait()

            c_base = i * _CHB

            def red(t, acc):
                v = idx_v[pl.ds(c_base + t * _LANES, _LANES)]
                return acc + jnp.where(v == 0, jnp.int32(1), jnp.int32(0))

            acc = lax.fori_loop(
                0, _CHB // _LANES, red, jnp.zeros((_LANES,), jnp.int32)
            )
            nzero = _lane_sum(acc)

            @pl.when(nzero > 0)
            def _(rows=rows, c_base=c_base, i=i):
                _zero_fixup(idx_v, rows, c_base, base + i * _CHB)

            @pl.when(k >= 1)
            def _(i=i, tr=tr, osem=osem):
                for cp in out_cps(i - 2, tr, osem):
                    cp.wait()

            transpose_unit(rows, tr)

            for cp in out_cps(i, tr, osem):
                cp.start()

            @pl.when(k < (_UPW // 2) - 1)
            def _(i=i, rows=rows, gsem=gsem):
                gather_cp(i + 2, rows, gsem).start()
        return 0

    lax.fori_loop(0, _UPW // 2, pair, 0)

    for cp in out_cps(_UPW - 2, tr0, os0):
        cp.wait()
    for cp in out_cps(_UPW - 1, tr1, os1):
        cp.wait()


def kernel(x, table):
    table_t = table.T
    tailp = jnp.pad(table[_VT * 128 :], ((0, 0), (0, 128 - _D))).reshape(-1)
    scratch = _table_rowmajor(table_t, tailp)
    table2 = scratch.reshape(2 * _V, _D)
    xt = x.T.reshape(_N).astype(jnp.int32)
    y = _emb_gather(xt, table2)
    y5 = y.reshape(_L, 8, _BT, 8, 128)
    return y5.transpose(2, 4, 0, 1, 3).reshape(_B, _L, _D)


# speedup vs baseline: 1.6411x; 1.4343x over previous
import functools

import jax
import jax.numpy as jnp
from jax import lax
from jax.experimental import pallas as pl
from jax.experimental.pallas import tpu as pltpu
from jax.experimental.pallas import tpu_sc as plsc

_B = 4096
_L = 200
_D = 64
_V = 1000000
_N = _B * _L
_NC = 2
_NS = 16
_NW = _NC * _NS
_LANES = 16

_VT = _V // 128
_ASLOT = 246

_CHB = 256
_UPW = _N // _NW // _CHB
_BT = _B // 128
_PW = _N // _NW


def _lane_sum(vec):
    s = vec[0]
    for i in range(1, _LANES):
        s = s + vec[i]
    return s


_mesh = plsc.VectorSubcoreMesh(core_axis_name="c", subcore_axis_name="s")


@functools.partial(
    pl.kernel,
    mesh=_mesh,
    out_type=jax.ShapeDtypeStruct((_V * 128,), jnp.float32),
    scratch_types=[
        pltpu.VMEM((2, _D, 128), jnp.float32),
        pltpu.VMEM((_D * 129 + 16,), jnp.float32),
        pltpu.VMEM((128 * 128,), jnp.float32),
        pltpu.VMEM((128 * 128,), jnp.float32),
        pltpu.VMEM((64 * 128,), jnp.float32),
        pltpu.SemaphoreType.DMA,
        pltpu.SemaphoreType.DMA,
        pltpu.SemaphoreType.DMA,
        pltpu.SemaphoreType.DMA,
    ],
    compiler_params=pltpu.CompilerParams(
        use_tc_tiling_on_sc=True, needs_layout_passes=False
    ),
)
def _table_rowmajor(
    tt_hbm, tail_hbm, out_hbm, inb, sk, outb0, outb1, tailv, is0, is1, os0, os1
):
    wid = lax.axis_index("s") * _NC + lax.axis_index("c")
    lane = lax.iota(jnp.int32, _LANES)
    rk129 = [(lane + k * 16) * 129 for k in range(4)]

    def ua(slot):
        return jnp.minimum(slot * _NW + wid, _VT - 1)

    def fire_in(slot, buf):
        vt = ua(slot)
        return [
            pltpu.make_async_copy(
                tt_hbm.at[:, pl.ds(vt * 128, 128)],
                inb.at[buf],
                is0 if buf == 0 else is1,
            )
        ]

    def transpose_unit(buf, outb):
        def d_body(dq, _):
            for dd in range(4):
                d = dq * 4 + dd
                db = d * 129
                for g in range(8):
                    v = inb[buf, d, pl.ds(g * 16, 16)]
                    plsc.store_scatter(sk, [lane + (db + g * 16)], v)
            return 0

        lax.fori_loop(0, _D // 4, d_body, 0)

        def v_body(vq, _):
            for vv in range(4):
                v = vq * 4 + vv
                for k in range(4):
                    r = plsc.load_gather(sk, [rk129[k] + v])
                    outb[pl.ds(v * 128 + k * 16, 16)] = r
            return 0

        lax.fori_loop(0, 32, v_body, 0)

    @pl.when(wid == 0)
    def _():
        pltpu.sync_copy(tail_hbm, tailv)
        pltpu.sync_copy(tailv, out_hbm.at[pl.ds(_VT * 16384, 64 * 128)])

    for cp in fire_in(0, 0):
        cp.start()
    for cp in fire_in(1, 1):
        cp.start()

    def pair(k, _):
        for buf, outb, osem in ((0, outb0, os0), (1, outb1, os1)):
            slot = 2 * k + buf
            vt = ua(slot)
            for cp in fire_in(slot, buf):
                cp.wait()
            transpose_unit(buf, outb)

            @pl.when(k >= 1)
            def _(outb=outb, osem=osem, slot=slot):
                pv = ua(slot - 2)
                pltpu.make_async_copy(
                    outb, out_hbm.at[pl.ds(pv * 16384, 16384)], osem
                ).wait()

            pltpu.async_copy(outb, out_hbm.at[pl.ds(vt * 16384, 16384)], osem)

            @pl.when(k < (_ASLOT // 2) - 1)
            def _(slot=slot, buf=buf):
                for cp in fire_in(slot + 2, buf):
                    cp.start()
        return 0

    lax.fori_loop(0, _ASLOT // 2, pair, 0)

    pltpu.make_async_copy(
        outb0, out_hbm.at[pl.ds(ua(_ASLOT - 2) * 16384, 16384)], os0
    ).wait()
    pltpu.make_async_copy(
        outb1, out_hbm.at[pl.ds(ua(_ASLOT - 1) * 16384, 16384)], os1
    ).wait()


def _zero_fixup(idx_v, rows, c_base, flat_base):
    lane = lax.iota(jnp.int32, _LANES)
    zeros = jnp.zeros((_LANES,), jnp.float32)

    def group(r, _):
        start = c_base + r * _LANES
        v = idx_v[pl.ds(start, _LANES)]
        pos = flat_base + r * _LANES + lane
        m = (v == 0) & (pos >= _B)
        hit = _lane_sum(jnp.where(m, jnp.int32(1), jnp.int32(0)))

        @pl.when(hit > 0)
        def _():
            for j in range(_LANES):
                vj = v[j]
                p = flat_base + r * _LANES + j

                @pl.when((vj == 0) & (p >= _B))
                def _(j=j):
                    row = r * _LANES + j
                    for k in range(_D // _LANES):
                        rows[row, pl.ds(k * _LANES, _LANES)] = zeros

        return 0

    lax.fori_loop(0, _CHB // _LANES, group, 0)


@functools.partial(
    pl.kernel,
    mesh=_mesh,
    out_type=jax.ShapeDtypeStruct((_L * 8 * _BT * 8, 128), jnp.float32),
    scratch_types=[
        pltpu.VMEM((_PW,), jnp.int32),
        pltpu.VMEM((_CHB, _D), jnp.float32),
        pltpu.VMEM((_CHB, _D), jnp.float32),
        pltpu.VMEM((_D, 257), jnp.float32),
        pltpu.VMEM((_D, 257), jnp.float32),
        pltpu.SemaphoreType.DMA,
        pltpu.SemaphoreType.DMA,
        pltpu.SemaphoreType.DMA,
        pltpu.SemaphoreType.DMA,
    ],
    compiler_params=pltpu.CompilerParams(
        use_tc_tiling_on_sc=False, needs_layout_passes=False
    ),
)
def _emb_gather(
    idx_hbm, table_hbm, out_hbm, idx_v, rows0, rows1, tr0, tr1, gs0, gs1, os0, os1
):
    wid = lax.axis_index("s") * _NC + lax.axis_index("c")
    base = wid * _PW
    lane = lax.iota(jnp.int32, _LANES)
    rowsk = [lane + k * 16 for k in range(4)]

    pltpu.sync_copy(idx_hbm.at[pl.ds(base, _PW)], idx_v)

    def dbl(i, _):
        v = idx_v[pl.ds(i * _LANES, _LANES)]
        idx_v[pl.ds(i * _LANES, _LANES)] = v + v
        return 0

    lax.fori_loop(0, _PW // _LANES, dbl, 0)

    def gather_cp(i, rows, sem):
        return pltpu.make_async_copy(
            table_hbm.at[idx_v.at[pl.ds(i * _CHB, _CHB)]], rows, sem
        )

    def out_cps(i, tr, osem):
        u = wid * _UPW + i
        l = u // 16
        bc = lax.rem(u, 16)
        cps = []
        for g in range(8):
            for bt in range(2):
                r0 = ((l * 8 + g) * _BT + bc * 2 + bt) * 8
                cps.append(
                    pltpu.make_async_copy(
                        tr.at[pl.ds(g * 8, 8), pl.ds(bt * 128, 128)],
                        out_hbm.at[pl.ds(r0, 8), :],
                        osem,
                    )
                )
        return cps

    def transpose_unit(rows, tr):
        def rbody(rq, _):
            for rr in range(4):
                r = rq * 4 + rr
                colv = jnp.full((_LANES,), r, jnp.int32)
                for k in range(4):
                    v = rows[r, pl.ds(k * 16, 16)]
                    plsc.store_scatter(tr, [rowsk[k], colv], v)
            return 0

        lax.fori_loop(0, _CHB // 4, rbody, 0)

    gather_cp(0, rows0, gs0).start()
    gather_cp(1, rows1, gs1).start()

    def pair(k, _):
        for buf, rows, tr, gsem, osem in (
            (0, rows0, tr0, gs0, os0),
            (1, rows1, tr1, gs1, os1),
        ):
            i = 2 * k + buf
            gather_cp(i, rows, gsem).wait()

            c_base = i * _CHB

            def red(t, acc):
                v = idx_v[pl.ds(c_base + t * _LANES, _LANES)]
                return acc + jnp.where(v == 0, jnp.int32(1), jnp.int32(0))

            acc = lax.fori_loop(
                0, _CHB // _LANES, red, jnp.zeros((_LANES,), jnp.int32)
            )
            nzero = _lane_sum(acc)

            @pl.when(nzero > 0)
            def _(rows=rows, c_base=c_base, i=i):
                _zero_fixup(idx_v, rows, c_base, base + i * _CHB)

            @pl.when(k >= 1)
            def _(i=i, tr=tr, osem=osem):
                for cp in out_cps(i - 2, tr, osem):
                    cp.wait()

            transpose_unit(rows, tr)

            for cp in out_cps(i, tr, osem):
                cp.start()

            @pl.when(k < (_UPW // 2) - 1)
            def _(i=i, rows=rows, gsem=gsem):
                gather_cp(i + 2, rows, gsem).start()
        return 0

    lax.fori_loop(0, _UPW // 2, pair, 0)

    for cp in out_cps(_UPW - 2, tr0, os0):
        cp.wait()
    for cp in out_cps(_UPW - 1, tr1, os1):
        cp.wait()


def kernel(x, table):
    table_t = table.T
    tailp = jnp.pad(table[_VT * 128 :], ((0, 0), (0, 128 - _D))).reshape(-1)
    scratch = _table_rowmajor(table_t, tailp)
    table2 = scratch.reshape(2 * _V, _D)
    xt = x.T.reshape(_N).astype(jnp.int32)
    y = _emb_gather(xt, table2)
    y5 = y.reshape(_L, 8, _BT, 8, 128)
    return y5.transpose(2, 4, 0, 1, 3).reshape(_B, _L, _D)


# speedup vs baseline: 2.3344x; 1.4225x over previous
import functools

import jax
import jax.numpy as jnp
from jax import lax
from jax.experimental import pallas as pl
from jax.experimental.pallas import tpu as pltpu
from jax.experimental.pallas import tpu_sc as plsc

_B = 4096
_L = 200
_D = 64
_V = 1000000
_N = _B * _L
_NC = 2
_NS = 16
_NW = _NC * _NS
_LANES = 16

_VT = _V // 128
_ASLOT = 246

_CHB = 256
_UPW = _N // _NW // _CHB
_BT = _B // 128
_PW = _N // _NW


def _lane_sum(vec):
    s = vec[0]
    for i in range(1, _LANES):
        s = s + vec[i]
    return s


_mesh = plsc.VectorSubcoreMesh(core_axis_name="c", subcore_axis_name="s")


@functools.partial(
    pl.kernel,
    mesh=_mesh,
    out_type=jax.ShapeDtypeStruct((_V * 128,), jnp.float32),
    scratch_types=[
        pltpu.VMEM((2, _D, 128), jnp.float32),
        pltpu.VMEM((_D * 129 + 16,), jnp.float32),
        pltpu.VMEM((128 * 128,), jnp.float32),
        pltpu.VMEM((128 * 128,), jnp.float32),
        pltpu.VMEM((64 * 128,), jnp.float32),
        pltpu.SemaphoreType.DMA,
        pltpu.SemaphoreType.DMA,
        pltpu.SemaphoreType.DMA,
        pltpu.SemaphoreType.DMA,
    ],
    compiler_params=pltpu.CompilerParams(
        use_tc_tiling_on_sc=True, needs_layout_passes=False
    ),
)
def _table_rowmajor(
    tt_hbm, tail_hbm, out_hbm, inb, sk, outb0, outb1, tailv, is0, is1, os0, os1
):
    wid = lax.axis_index("s") * _NC + lax.axis_index("c")
    lane = lax.iota(jnp.int32, _LANES)
    rk129 = [(lane + k * 16) * 129 for k in range(4)]

    def ua(slot):
        return jnp.minimum(slot * _NW + wid, _VT - 1)

    def fire_in(slot, buf):
        vt = ua(slot)
        return [
            pltpu.make_async_copy(
                tt_hbm.at[:, pl.ds(vt * 128, 128)],
                inb.at[buf],
                is0 if buf == 0 else is1,
            )
        ]

    def transpose_unit(buf, outb):
        def d_body(dq, _):
            for dd in range(4):
                d = dq * 4 + dd
                db = d * 129
                for g in range(8):
                    v = inb[buf, d, pl.ds(g * 16, 16)]
                    plsc.store_scatter(sk, [lane + (db + g * 16)], v)
            return 0

        lax.fori_loop(0, _D // 4, d_body, 0)

        def v_body(vq, _):
            for vv in range(4):
                v = vq * 4 + vv
                for k in range(4):
                    r = plsc.load_gather(sk, [rk129[k] + v])
                    outb[pl.ds(v * 128 + k * 16, 16)] = r
            return 0

        lax.fori_loop(0, 32, v_body, 0)

    @pl.when(wid == 0)
    def _():
        pltpu.sync_copy(tail_hbm, tailv)
        pltpu.sync_copy(tailv, out_hbm.at[pl.ds(_VT * 16384, 64 * 128)])

    for cp in fire_in(0, 0):
        cp.start()
    for cp in fire_in(1, 1):
        cp.start()

    def pair(k, _):
        for buf, outb, osem in ((0, outb0, os0), (1, outb1, os1)):
            slot = 2 * k + buf
            vt = ua(slot)
            for cp in fire_in(slot, buf):
                cp.wait()
            transpose_unit(buf, outb)

            @pl.when(k >= 1)
            def _(outb=outb, osem=osem, slot=slot):
                pv = ua(slot - 2)
                pltpu.make_async_copy(
                    outb, out_hbm.at[pl.ds(pv * 16384, 16384)], osem
                ).wait()

            pltpu.async_copy(outb, out_hbm.at[pl.ds(vt * 16384, 16384)], osem)

            @pl.when(k < (_ASLOT // 2) - 1)
            def _(slot=slot, buf=buf):
                for cp in fire_in(slot + 2, buf):
                    cp.start()
        return 0

    lax.fori_loop(0, _ASLOT // 2, pair, 0)

    pltpu.make_async_copy(
        outb0, out_hbm.at[pl.ds(ua(_ASLOT - 2) * 16384, 16384)], os0
    ).wait()
    pltpu.make_async_copy(
        outb1, out_hbm.at[pl.ds(ua(_ASLOT - 1) * 16384, 16384)], os1
    ).wait()


def _zero_fixup(idx_v, rows, c_base, flat_base):
    lane = lax.iota(jnp.int32, _LANES)
    zeros = jnp.zeros((_LANES,), jnp.float32)

    def group(r, _):
        start = c_base + r * _LANES
        v = idx_v[pl.ds(start, _LANES)]
        pos = flat_base + r * _LANES + lane
        m = (v == 0) & (pos >= _B)
        hit = _lane_sum(jnp.where(m, jnp.int32(1), jnp.int32(0)))

        @pl.when(hit > 0)
        def _():
            for j in range(_LANES):
                vj = v[j]
                p = flat_base + r * _LANES + j

                @pl.when((vj == 0) & (p >= _B))
                def _(j=j):
                    row = r * _LANES + j
                    for k in range(_D // _LANES):
                        rows[row, pl.ds(k * _LANES, _LANES)] = zeros

        return 0

    lax.fori_loop(0, _CHB // _LANES, group, 0)


@functools.partial(
    pl.kernel,
    mesh=_mesh,
    out_type=jax.ShapeDtypeStruct((_L * 8 * _BT * 8, 128), jnp.float32),
    scratch_types=[
        pltpu.VMEM((_PW,), jnp.int32),
        pltpu.VMEM((_CHB, _D), jnp.float32),
        pltpu.VMEM((_CHB, _D), jnp.float32),
        pltpu.VMEM((_D, 257), jnp.float32),
        pltpu.VMEM((_D, 257), jnp.float32),
        pltpu.SemaphoreType.DMA,
        pltpu.SemaphoreType.DMA,
        pltpu.SemaphoreType.DMA,
        pltpu.SemaphoreType.DMA,
    ],
    compiler_params=pltpu.CompilerParams(
        use_tc_tiling_on_sc=False, needs_layout_passes=False
    ),
)
def _emb_gather(
    idx_hbm, table_hbm, out_hbm, idx_v, rows0, rows1, tr0, tr1, gs0, gs1, os0, os1
):
    wid = lax.axis_index("s") * _NC + lax.axis_index("c")
    base = wid * _PW
    lane = lax.iota(jnp.int32, _LANES)
    rowsk = [lane + k * 16 for k in range(4)]

    pltpu.sync_copy(idx_hbm.at[pl.ds(base, _PW)], idx_v)

    def dbl(i, _):
        v = idx_v[pl.ds(i * _LANES, _LANES)]
        idx_v[pl.ds(i * _LANES, _LANES)] = v + v
        return 0

    lax.fori_loop(0, _PW // _LANES, dbl, 0)

    def gather_cp(i, rows, sem):
        return pltpu.make_async_copy(
            table_hbm.at[idx_v.at[pl.ds(i * _CHB, _CHB)]], rows, sem
        )

    def out_cps(i, tr, osem):
        u = wid * _UPW + i
        l = u // 16
        bc = lax.rem(u, 16)
        cps = []
        for g in range(8):
            for bt in range(2):
                r0 = ((l * 8 + g) * _BT + bc * 2 + bt) * 8
                cps.append(
                    pltpu.make_async_copy(
                        tr.at[pl.ds(g * 8, 8), pl.ds(bt * 128, 128)],
                        out_hbm.at[pl.ds(r0, 8), :],
                        osem,
                    )
                )
        return cps

    def transpose_unit(rows, tr):
        def rbody(rq, _):
            for rr in range(4):
                r = rq * 4 + rr
                colv = jnp.full((_LANES,), r, jnp.int32)
                for k in range(4):
                    v = rows[r, pl.ds(k * 16, 16)]
                    plsc.store_scatter(tr, [rowsk[k], colv], v)
            return 0

        lax.fori_loop(0, _CHB // 4, rbody, 0)

    gather_cp(0, rows0, gs0).start()
    gather_cp(1, rows1, gs1).start()

    def pair(k, _):
        for buf, rows, tr, gsem, osem in (
            (0, rows0, tr0, gs0, os0),
            (1, rows1, tr1, gs1, os1),
        ):
            i = 2 * k + buf
            gather_cp(i, rows, gsem).wait()

            c_base = i * _CHB

            def red(t, acc):
                v = idx_v[pl.ds(c_base + t * _LANES, _LANES)]
                return acc + jnp.where(v == 0, jnp.int32(1), jnp.int32(0))

            acc = lax.fori_loop(
                0, _CHB // _LANES, red, jnp.zeros((_LANES,), jnp.int32)
            )
            nzero = _lane_sum(acc)

            @pl.when(nzero > 0)
            def _(rows=rows, c_base=c_base, i=i):
                _zero_fixup(idx_v, rows, c_base, base + i * _CHB)

            @pl.when(k >= 1)
            def _(i=i, tr=tr, osem=osem):
                for cp in out_cps(i - 2, tr, osem):
                    cp.wait()

            transpose_unit(rows, tr)

            for cp in out_cps(i, tr, osem):
                cp.start()

            @pl.when(k < (_UPW // 2) - 1)
            def _(i=i, rows=rows, gsem=gsem):
                gather_cp(i + 2, rows, gsem).start()
        return 0

    lax.fori_loop(0, _UPW // 2, pair, 0)

    for cp in out_cps(_UPW - 2, tr0, os0):
        cp.wait()
    for cp in out_cps(_UPW - 1, tr1, os1):
        cp.wait()


def _flat_view(tp):
    return pl.pallas_call(
        lambda i_ref, o_ref: None,
        out_shape=jax.ShapeDtypeStruct((_V * 128,), jnp.float32),
        in_specs=[pl.BlockSpec(memory_space=pl.ANY)],
        out_specs=pl.BlockSpec(memory_space=pl.ANY),
        input_output_aliases={0: 0},
    )(tp)


def kernel(x, table):
    table2 = jnp.pad(table, ((0, 0), (0, 128 - _D))).reshape(2 * _V, _D)
    xt = x.T.reshape(_N).astype(jnp.int32)
    y = _emb_gather(xt, table2)
    y5 = y.reshape(_L, 8, _BT, 8, 128)
    return y5.transpose(2, 4, 0, 1, 3).reshape(_B, _L, _D)
